# baseline (speedup 1.0000x reference)
import functools

import jax
import jax.numpy as jnp
from jax import lax
from jax.experimental import pallas as pl
from jax.experimental.pallas import tpu as pltpu
from jax.experimental.pallas import tpu_sc as plsc

MAXV = 1024
NC = 2
NS = 16
NW = NC * NS
NDG = 4
NBG = NW // NDG
NB = 16


def _make_kernel(B, D, P):
    d_per_g = D // NDG
    b_per_g = B // NBG
    n_chunks = b_per_g // NB
    mesh = plsc.VectorSubcoreMesh(core_axis_name="c", subcore_axis_name="s")

    @functools.partial(
        pl.kernel,
        mesh=mesh,
        out_type=jax.ShapeDtypeStruct((B, D, 8, 8), jnp.float32),
        compiler_params=pltpu.CompilerParams(use_tc_tiling_on_sc=False,
                                             needs_layout_passes=False),
        scratch_types=[
            pltpu.VMEM((d_per_g // 2 * 3, MAXV), jnp.int32),
            pltpu.VMEM((2, NB, 3, 8, 8), jnp.int32),
            pltpu.VMEM((2, NB, d_per_g, 8, 8), jnp.float32),
            pltpu.SemaphoreType.DMA,
            pltpu.SemaphoreType.DMA,
            pltpu.SemaphoreType.DMA,
            pltpu.SemaphoreType.DMA,
        ],
    )
    def k(idx_hbm, tbl_hbm, out_hbm, tbl_v, idx_v, out_v,
          sin0, sin1, sout0, sout1):
        cid = lax.axis_index("c")
        sid = lax.axis_index("s")
        wid = sid * NC + cid
        dg = wid % NDG
        bg = wid // NDG
        sins = (sin0, sin1)
        souts = (sout0, sout1)
        ji = lax.iota(jnp.int32, 16)
        widx = ji & 7
        hrow = lax.shift_right_logical(ji, 3)
        hidxs = [hrow + 2 * ch for ch in range(P // 16)]

        def idx_copy(g, buf):
            b0 = bg * b_per_g + g * NB
            return pltpu.make_async_copy(
                idx_hbm.at[pl.ds(b0, NB)], idx_v.at[buf], sins[buf])

        def out_copy(g, buf):
            b0 = bg * b_per_g + g * NB
            return pltpu.make_async_copy(
                out_v.at[buf],
                out_hbm.at[pl.ds(b0, NB), pl.ds(dg * d_per_g, d_per_g)],
                souts[buf])

        d_pairs = d_per_g // 2
        pltpu.sync_copy(tbl_hbm.at[pl.ds(dg * (d_pairs * 3), d_pairs * 3)],
                        tbl_v)
        idx_copy(0, 0).start()
        idx_copy(1, 1).start()

        def pair_body(p, carry):
            for buf in range(2):
                g = p * 2 + buf
                idx_copy(g, buf).wait()

                @pl.when(g >= 2)
                def _():
                    out_copy(g - 2, buf).wait()

                def b_body(bl, carry2):
                    ivs = [[plsc.load_gather(idx_v.at[buf, bl, c],
                                             [hidxs[ch], widx])
                            for ch in range(P // 16)] for c in range(3)]

                    def gather_trio(dp, ch):
                        return [plsc.load_gather(tbl_v.at[dp * 3 + c],
                                                 [ivs[c][ch]])
                                for c in range(3)]

                    def emit_trio(dp, ch, ws):
                        w0, w1, w2 = ws
                        acc = (plsc.bitcast(w0, jnp.bfloat16)
                               + plsc.bitcast(w1, jnp.bfloat16))
                        acc = acc + plsc.bitcast(w2, jnp.bfloat16)
                        lo, hi = plsc.unpack(
                            acc, format=plsc.PackFormat.INTERLEAVED)
                        plsc.store_scatter(out_v.at[buf, bl, 2 * dp],
                                           [hidxs[ch], widx], lo)
                        plsc.store_scatter(out_v.at[buf, bl, 2 * dp + 1],
                                           [hidxs[ch], widx], hi)

                    nch = P // 16
                    trios = [(dp, ch) for dp in range(d_pairs)
                             for ch in range(nch)]
                    depth = 4
                    pending = [gather_trio(*trios[j]) for j in range(depth)]
                    for j, (dp, ch) in enumerate(trios):
                        if j + depth < len(trios):
                            pending.append(gather_trio(*trios[j + depth]))
                        emit_trio(dp, ch, pending.pop(0))
                    return carry2

                lax.fori_loop(0, NB, b_body, 0)

                @pl.when(g + 2 < n_chunks)
                def _():
                    idx_copy(g + 2, buf).start()

                out_copy(g, buf).start()
            return carry

        lax.fori_loop(0, n_chunks // 2, pair_body, 0)
        out_copy(n_chunks - 2, 0).wait()
        out_copy(n_chunks - 1, 1).wait()

    return k


def kernel(inputs, table):
    B, C, H, W = inputs.shape
    V3, D = table.shape
    P = H * W
    tbf = table.astype(jnp.bfloat16)
    tw = jax.lax.bitcast_convert_type(
        tbf.reshape(V3, D // 2, 2), jnp.int32)
    twr = tw.T.reshape(D // 2 * C, MAXV)
    return _make_kernel(B, D, P)(inputs, twr)


# speedup vs baseline: 3.1351x; 3.1351x over previous
import functools

import jax
import jax.numpy as jnp
from jax import lax
from jax.experimental import pallas as pl
from jax.experimental.pallas import tpu as pltpu
from jax.experimental.pallas import tpu_sc as plsc

MAXV = 1024
NC = 2
NS = 16
NW = NC * NS
NDG = 4
NBG = NW // NDG
NB = 16


def _make_kernel(B, D, P):
    d_per_g = D // NDG
    b_per_g = B // NBG
    n_chunks = b_per_g // NB
    mesh = plsc.VectorSubcoreMesh(core_axis_name="c", subcore_axis_name="s")

    @functools.partial(
        pl.kernel,
        mesh=mesh,
        out_type=jax.ShapeDtypeStruct((B, P, D), jnp.float32),
        compiler_params=pltpu.CompilerParams(use_tc_tiling_on_sc=False,
                                             needs_layout_passes=False),
        scratch_types=[
            pltpu.VMEM((d_per_g // 2 * 3, MAXV), jnp.int32),
            pltpu.VMEM((2, NB, 3, P), jnp.int32),
            pltpu.VMEM((2, NB, P, d_per_g), jnp.float32),
            pltpu.SemaphoreType.DMA,
            pltpu.SemaphoreType.DMA,
            pltpu.SemaphoreType.DMA,
            pltpu.SemaphoreType.DMA,
        ],
    )
    def k(idx_hbm, tbl_hbm, out_hbm, tbl_v, idx_v, out_v,
          sin0, sin1, sout0, sout1):
        cid = lax.axis_index("c")
        sid = lax.axis_index("s")
        wid = sid * NC + cid
        dg = wid % NDG
        bg = wid // NDG
        sins = (sin0, sin1)
        souts = (sout0, sout1)
        ji = lax.iota(jnp.int32, 16)
        pidxs = [ji + 16 * ch for ch in range(P // 16)]

        def idx_copy(g, buf):
            b0 = bg * b_per_g + g * NB
            return pltpu.make_async_copy(
                idx_hbm.at[pl.ds(b0, NB)], idx_v.at[buf], sins[buf])

        def out_copy(g, buf):
            b0 = bg * b_per_g + g * NB
            return pltpu.make_async_copy(
                out_v.at[buf],
                out_hbm.at[pl.ds(b0, NB), :, pl.ds(dg * d_per_g, d_per_g)],
                souts[buf])

        d_pairs = d_per_g // 2
        pltpu.sync_copy(tbl_hbm.at[pl.ds(dg * (d_pairs * 3), d_pairs * 3)],
                        tbl_v)
        idx_copy(0, 0).start()
        idx_copy(1, 1).start()

        def pair_body(p, carry):
            for buf in range(2):
                g = p * 2 + buf
                idx_copy(g, buf).wait()

                @pl.when(g >= 2)
                def _():
                    out_copy(g - 2, buf).wait()

                def b_body(bl, carry2):
                    ivs = [[idx_v[buf, bl, c, pl.ds(ch * 16, 16)]
                            for ch in range(P // 16)] for c in range(3)]

                    def gather_trio(dp, ch):
                        return [plsc.load_gather(tbl_v.at[dp * 3 + c],
                                                 [ivs[c][ch]])
                                for c in range(3)]

                    def emit_trio(dp, ch, ws):
                        w0, w1, w2 = ws
                        acc = (plsc.bitcast(w0, jnp.bfloat16)
                               + plsc.bitcast(w1, jnp.bfloat16))
                        acc = acc + plsc.bitcast(w2, jnp.bfloat16)
                        lo, hi = plsc.unpack(
                            acc, format=plsc.PackFormat.INTERLEAVED)
                        dcol = jnp.full((16,), 2 * dp, jnp.int32)
                        plsc.store_scatter(out_v.at[buf, bl],
                                           [pidxs[ch], dcol], lo)
                        plsc.store_scatter(out_v.at[buf, bl],
                                           [pidxs[ch], dcol + 1], hi)

                    nch = P // 16
                    trios = [(dp, ch) for dp in range(d_pairs)
                             for ch in range(nch)]
                    depth = 4
                    pending = [gather_trio(*trios[j]) for j in range(depth)]
                    for j, (dp, ch) in enumerate(trios):
                        if j + depth < len(trios):
                            pending.append(gather_trio(*trios[j + depth]))
                        emit_trio(dp, ch, pending.pop(0))
                    return carry2

                lax.fori_loop(0, NB, b_body, 0)

                @pl.when(g + 2 < n_chunks)
                def _():
                    idx_copy(g + 2, buf).start()

                out_copy(g, buf).start()
            return carry

        lax.fori_loop(0, n_chunks // 2, pair_body, 0)
        out_copy(n_chunks - 2, 0).wait()
        out_copy(n_chunks - 1, 1).wait()

    return k


def kernel(inputs, table):
    B, C, H, W = inputs.shape
    V3, D = table.shape
    P = H * W
    tbf = table.astype(jnp.bfloat16)
    tw = jax.lax.bitcast_convert_type(
        tbf.reshape(V3, D // 2, 2), jnp.int32)
    twr = tw.T.reshape(D // 2 * C, MAXV)
    idx = inputs.reshape(B, C, P)
    out = _make_kernel(B, D, P)(idx, twr)
    return jnp.transpose(out.reshape(B, H, W, D), (0, 3, 1, 2))


# speedup vs baseline: 7.2515x; 2.3130x over previous
import functools

import jax
import jax.numpy as jnp
from jax import lax
from jax.experimental import pallas as pl
from jax.experimental.pallas import tpu as pltpu
from jax.experimental.pallas import tpu_sc as plsc

MAXV = 1024
NC = 2
NS = 16
NW = NC * NS
NB = 2


def _make_kernel(B, D, P):
    b_per_w = B // NW
    n_chunks = b_per_w // NB
    nrows = NB * 3 * P
    wpr = D // 2
    mesh = plsc.VectorSubcoreMesh(core_axis_name="c", subcore_axis_name="s")

    @functools.partial(
        pl.kernel,
        mesh=mesh,
        out_type=jax.ShapeDtypeStruct((B, P, D), jnp.float32),
        compiler_params=pltpu.CompilerParams(use_tc_tiling_on_sc=False,
                                             needs_layout_passes=False),
        scratch_types=[
            pltpu.VMEM((2, NB, 3, P), jnp.int32),
            pltpu.VMEM((2, nrows), jnp.int32),
            pltpu.VMEM((2, nrows, wpr), jnp.int32),
            pltpu.VMEM((2, NB, P, D), jnp.float32),
            pltpu.SemaphoreType.DMA,
            pltpu.SemaphoreType.DMA,
            pltpu.SemaphoreType.DMA,
            pltpu.SemaphoreType.DMA,
            pltpu.SemaphoreType.DMA,
            pltpu.SemaphoreType.DMA,
        ],
    )
    def k(idx_hbm, tbl_hbm, out_hbm, idx_v, list_v, rows_v, out_v,
          si0, si1, sg0, sg1, so0, so1):
        cid = lax.axis_index("c")
        sid = lax.axis_index("s")
        wid = sid * NC + cid
        sis = (si0, si1)
        sgs = (sg0, sg1)
        sos = (so0, so1)

        ji = lax.iota(jnp.int32, 16)
        p3c = [[(ji + 16 * ch) * 3 + c for c in range(3)]
               for ch in range(P // 16)]
        evens = [ji * 2 + 32 * kk for kk in range(wpr // 16)]

        def idx_copy(g, buf):
            b0 = wid * b_per_w + g * NB
            return pltpu.make_async_copy(
                idx_hbm.at[pl.ds(b0, NB)], idx_v.at[buf], sis[buf])

        def row_gather(buf):
            return pltpu.make_async_copy(
                tbl_hbm.at[list_v.at[buf]], rows_v.at[buf], sgs[buf])

        def out_copy(g, buf):
            b0 = wid * b_per_w + g * NB
            return pltpu.make_async_copy(
                out_v.at[buf], out_hbm.at[pl.ds(b0, NB)], sos[buf])

        def build_list(buf):
            for bl in range(NB):
                dst = list_v.at[buf, pl.ds(bl * 3 * P, 3 * P)]
                for ch in range(P // 16):
                    for c in range(3):
                        iv = idx_v[buf, bl, c, pl.ds(ch * 16, 16)]
                        if c:
                            iv = iv + (c * MAXV)
                        plsc.store_scatter(dst, [p3c[ch][c]], iv)

        idx_copy(0, 0).start()
        idx_copy(1, 1).start()
        idx_copy(0, 0).wait()
        build_list(0)
        row_gather(0).start()

        def pair_body(pp, carry):
          for buf in range(2):
            g = pp * 2 + buf
            nxt = 1 - buf
            row_gather(buf).wait()

            @pl.when(g + 1 < n_chunks)
            def _():
                idx_copy(g + 1, nxt).wait()
                build_list(nxt)
                row_gather(nxt).start()

            @pl.when(g + 2 < n_chunks)
            def _():
                idx_copy(g + 2, buf).start()

            @pl.when(g >= 2)
            def _():
                out_copy(g - 2, buf).wait()

            def b_body(bl, carry2):
                rbase = bl * 3 * P

                def row_words(p, c):
                    return [rows_v[buf, rbase + p * 3 + c, pl.ds(kk * 16, 16)]
                            for kk in range(wpr // 16)]

                def emit_pixel(p, ws):
                    dst = out_v.at[buf, bl, p]
                    for kk in range(wpr // 16):
                        acc = (plsc.bitcast(ws[0][kk], jnp.bfloat16)
                               + plsc.bitcast(ws[1][kk], jnp.bfloat16))
                        acc = acc + plsc.bitcast(ws[2][kk], jnp.bfloat16)
                        lo, hi = plsc.unpack(
                            acc, format=plsc.PackFormat.INTERLEAVED)
                        plsc.store_scatter(dst, [evens[kk]], lo)
                        plsc.store_scatter(dst, [evens[kk] + 1], hi)

                prev = [row_words(0, c) for c in range(3)]
                for p in range(1, P):
                    cur = [row_words(p, c) for c in range(3)]
                    emit_pixel(p - 1, prev)
                    prev = cur
                emit_pixel(P - 1, prev)
                return carry2

            lax.fori_loop(0, NB, b_body, 0)
            out_copy(g, buf).start()
          return carry

        lax.fori_loop(0, n_chunks // 2, pair_body, 0)
        out_copy(n_chunks - 2, 0).wait()
        out_copy(n_chunks - 1, 1).wait()

    return k


def kernel(inputs, table):
    B, C, H, W = inputs.shape
    V3, D = table.shape
    P = H * W
    tbf = table.astype(jnp.bfloat16)
    tw = jax.lax.bitcast_convert_type(
        tbf.reshape(V3, D // 2, 2), jnp.int32)
    idx = inputs.reshape(B, C, P)
    out = _make_kernel(B, D, P)(idx, tw)
    return jnp.transpose(out.reshape(B, H, W, D), (0, 3, 1, 2))


# speedup vs baseline: 7.2720x; 1.0028x over previous
import functools

import jax
import jax.numpy as jnp
from jax import lax
from jax.experimental import pallas as pl
from jax.experimental.pallas import tpu as pltpu
from jax.experimental.pallas import tpu_sc as plsc

MAXV = 1024
NC = 2
NS = 16
NW = NC * NS
NB = 2


def _make_kernel(B, D, P):
    b_per_w = B // NW
    n_chunks = b_per_w // NB
    nrows = NB * 3 * P
    wpr = D // 2
    mesh = plsc.VectorSubcoreMesh(core_axis_name="c", subcore_axis_name="s")

    @functools.partial(
        pl.kernel,
        mesh=mesh,
        out_type=jax.ShapeDtypeStruct((B, P, D), jnp.float32),
        compiler_params=pltpu.CompilerParams(use_tc_tiling_on_sc=False,
                                             needs_layout_passes=False),
        scratch_types=[
            pltpu.VMEM((2, NB, 3, P), jnp.int32),
            pltpu.VMEM((2, nrows), jnp.int32),
            pltpu.VMEM((2, nrows, wpr), jnp.int32),
            pltpu.VMEM((2, NB, P, D), jnp.float32),
            pltpu.SemaphoreType.DMA,
            pltpu.SemaphoreType.DMA,
            pltpu.SemaphoreType.DMA,
            pltpu.SemaphoreType.DMA,
            pltpu.SemaphoreType.DMA,
            pltpu.SemaphoreType.DMA,
        ],
    )
    def k(idx_hbm, tbl_hbm, out_hbm, idx_v, list_v, rows_v, out_v,
          si0, si1, sg0, sg1, so0, so1):
        cid = lax.axis_index("c")
        sid = lax.axis_index("s")
        wid = sid * NC + cid
        sis = (si0, si1)
        sgs = (sg0, sg1)
        sos = (so0, so1)

        ji = lax.iota(jnp.int32, 16)
        p3c = [[(ji + 16 * ch) * 3 + c for c in range(3)]
               for ch in range(P // 16)]
        evens = [ji * 2 + 32 * kk for kk in range(wpr // 16)]

        def idx_copy(g, buf):
            b0 = wid * b_per_w + g * NB
            return pltpu.make_async_copy(
                idx_hbm.at[pl.ds(b0, NB)], idx_v.at[buf], sis[buf])

        def row_gather(buf):
            return pltpu.make_async_copy(
                tbl_hbm.at[list_v.at[buf]], rows_v.at[buf], sgs[buf])

        def out_copy(g, buf):
            b0 = wid * b_per_w + g * NB
            return pltpu.make_async_copy(
                out_v.at[buf], out_hbm.at[pl.ds(b0, NB)], sos[buf])

        def build_list(buf):
            for bl in range(NB):
                dst = list_v.at[buf, pl.ds(bl * 3 * P, 3 * P)]
                for ch in range(P // 16):
                    for c in range(3):
                        iv = idx_v[buf, bl, c, pl.ds(ch * 16, 16)]
                        if c:
                            iv = iv + (c * MAXV)
                        plsc.store_scatter(dst, [p3c[ch][c]], iv)

        idx_copy(0, 0).start()
        idx_copy(1, 1).start()
        idx_copy(0, 0).wait()
        build_list(0)
        row_gather(0).start()

        def pair_body(pp, carry):
          for buf in range(2):
            g = pp * 2 + buf
            nxt = 1 - buf

            @pl.when(g + 1 < n_chunks)
            def _():
                idx_copy(g + 1, nxt).wait()
                build_list(nxt)
                row_gather(nxt).start()

            @pl.when(g + 2 < n_chunks)
            def _():
                idx_copy(g + 2, buf).start()

            row_gather(buf).wait()

            @pl.when(g >= 2)
            def _():
                out_copy(g - 2, buf).wait()

            def b_body(bl, carry2):
                rbase = bl * 3 * P

                def row_words(p, c):
                    return [rows_v[buf, rbase + p * 3 + c, pl.ds(kk * 16, 16)]
                            for kk in range(wpr // 16)]

                def emit_pixel(p, ws):
                    dst = out_v.at[buf, bl, p]
                    for kk in range(wpr // 16):
                        acc = (plsc.bitcast(ws[0][kk], jnp.bfloat16)
                               + plsc.bitcast(ws[1][kk], jnp.bfloat16))
                        acc = acc + plsc.bitcast(ws[2][kk], jnp.bfloat16)
                        lo, hi = plsc.unpack(
                            acc, format=plsc.PackFormat.INTERLEAVED)
                        plsc.store_scatter(dst, [evens[kk]], lo)
                        plsc.store_scatter(dst, [evens[kk] + 1], hi)

                prev = [row_words(0, c) for c in range(3)]
                for p in range(1, P):
                    cur = [row_words(p, c) for c in range(3)]
                    emit_pixel(p - 1, prev)
                    prev = cur
                emit_pixel(P - 1, prev)
                return carry2

            lax.fori_loop(0, NB, b_body, 0)
            out_copy(g, buf).start()
          return carry

        lax.fori_loop(0, n_chunks // 2, pair_body, 0)
        out_copy(n_chunks - 2, 0).wait()
        out_copy(n_chunks - 1, 1).wait()

    return k


def kernel(inputs, table):
    B, C, H, W = inputs.shape
    V3, D = table.shape
    P = H * W
    tbf = table.astype(jnp.bfloat16)
    tw = jax.lax.bitcast_convert_type(
        tbf.reshape(V3, D // 2, 2), jnp.int32)
    idx = inputs.reshape(B, C, P)
    out = _make_kernel(B, D, P)(idx, tw)
    return jnp.transpose(out.reshape(B, H, W, D), (0, 3, 1, 2))
